# Initial kernel scaffold; baseline (speedup 1.0000x reference)
import jax
import jax.numpy as jnp
from jax.experimental import pallas as pl


def kernel(xyz, fea, knn_idx, ch_mask, body0_kp, body0_W, body0_g, body0_b, body1_kp, body1_W, body1_g, body1_b, ptm0_W, ptm0_g, ptm0_b, ptm1_kp, ptm1_W, ptm1_g, ptm1_b, ptm2_kp, ptm2_W, tail_W, tail_bias, tail_g, tail_bb):
    raise NotImplementedError("write your pallas kernel here")


# speedup vs baseline: 3.4625x; 3.4625x over previous
import functools

import jax
import jax.numpy as jnp
from jax import lax
from jax.experimental import pallas as pl
from jax.experimental.pallas import tpu as pltpu
from jax.experimental.pallas import tpu_sc as plsc

B_, N_, K_, CIN, COUT, KS, NL = 2, 4096, 16, 128, 128, 5, 2
BN = B_ * N_
M_ = BN * K_
RADIUS, TAU, EPS = 1.0, 1.0, 1e-5


def _sc_workers():
    try:
        info = plsc.get_sparse_core_info()
        return info.num_cores, info.num_subcores
    except Exception:
        return 2, 16


def _gather_rows(table, idx, chunk):
    _, d = table.shape
    (m,) = idx.shape
    nc, ns = _sc_workers()
    per_w = m // (nc * ns)
    nch = per_w // chunk
    mesh = plsc.VectorSubcoreMesh(core_axis_name="c", subcore_axis_name="s")

    @functools.partial(
        pl.kernel,
        out_type=jax.ShapeDtypeStruct((m, d), jnp.float32),
        mesh=mesh,
        scratch_types=[
            pltpu.VMEM((chunk,), jnp.int32),
            pltpu.VMEM((chunk, d), jnp.float32),
            pltpu.SemaphoreType.DMA,
        ],
        compiler_params=pltpu.CompilerParams(use_tc_tiling_on_sc=False),
    )
    def gk(table_hbm, idx_hbm, out_hbm, idx_v, rows_v, sem):
        wid = lax.axis_index("s") * nc + lax.axis_index("c")
        base = wid * per_w

        def body(i, carry):
            off = base + i * chunk
            pltpu.sync_copy(idx_hbm.at[pl.ds(off, chunk)], idx_v)
            pltpu.async_copy(table_hbm.at[idx_v], rows_v, sem).wait()
            pltpu.sync_copy(rows_v, out_hbm.at[pl.ds(off, chunk)])
            return carry

        lax.fori_loop(0, nch, body, 0)

    return gk(table, idx)


def _accum_stats(raw, sums_ref):
    @pl.when(pl.program_id(0) == 0)
    def _():
        sums_ref[...] = jnp.zeros_like(sums_ref)

    s1 = jnp.sum(raw, axis=0, keepdims=True)
    s2 = jnp.sum(raw * raw, axis=0, keepdims=True)
    sums_ref[...] = sums_ref[...] + jnp.concatenate([s1, s2], axis=0)


def _mm_stats_kernel(x_ref, w_ref, raw_ref, sums_ref):
    raw = jnp.dot(x_ref[...], w_ref[...], preferred_element_type=jnp.float32)
    raw_ref[...] = raw
    _accum_stats(raw, sums_ref)


def _mm_stats(x, w, tp):
    bn, cin = x.shape
    cout = w.shape[1]
    return pl.pallas_call(
        _mm_stats_kernel,
        grid=(bn // tp,),
        in_specs=[
            pl.BlockSpec((tp, cin), lambda i: (i, 0)),
            pl.BlockSpec((cin, cout), lambda i: (0, 0)),
        ],
        out_specs=[
            pl.BlockSpec((tp, cout), lambda i: (i, 0)),
            pl.BlockSpec((2, cout), lambda i: (0, 0)),
        ],
        out_shape=[
            jax.ShapeDtypeStruct((bn, cout), jnp.float32),
            jax.ShapeDtypeStruct((2, cout), jnp.float32),
        ],
    )(x, w)


def _kpconv_core(g_ref, nb_ref, ctr_ref, kp_ref, w_ref):
    nb = nb_ref[...]
    ctr = ctr_ref[...]
    rel = (nb - ctr[:, None, :]) * (1.0 / RADIUS)
    kp = kp_ref[...]
    infl = []
    for m in range(KS):
        d2 = jnp.sum((rel - kp[m][None, None, :]) ** 2, axis=-1)
        infl.append(jnp.maximum(1.0 - jnp.sqrt(d2 + 1e-12), 0.0))
    gg = g_ref[...]
    tp, _, c = gg.shape
    acc = [jnp.zeros((tp, c), jnp.float32) for _ in range(KS)]
    for k in range(K_):
        gk = gg[:, k, :]
        for m in range(KS):
            acc[m] = acc[m] + infl[m][:, k:k + 1] * gk
    cat = jnp.concatenate(acc, axis=1)
    return jnp.dot(cat, w_ref[...], preferred_element_type=jnp.float32)


def _kpconv_kernel(g_ref, nb_ref, ctr_ref, kp_ref, w_ref, raw_ref, sums_ref):
    raw = _kpconv_core(g_ref, nb_ref, ctr_ref, kp_ref, w_ref)
    raw_ref[...] = raw
    _accum_stats(raw, sums_ref)


def _kpconv_mask_kernel(g_ref, nb_ref, ctr_ref, kp_ref, w_ref, out_ref):
    raw = _kpconv_core(g_ref, nb_ref, ctr_ref, kp_ref, w_ref)
    out_ref[...] = jax.nn.sigmoid((raw[:, 1:2] - raw[:, 0:1]) / TAU)


def _kpconv_specs(tp, cin, cout, kp8, wf):
    in_specs = [
        pl.BlockSpec((tp, K_, cin), lambda i: (i, 0, 0)),
        pl.BlockSpec((tp, K_, 8), lambda i: (i, 0, 0)),
        pl.BlockSpec((tp, 8), lambda i: (i, 0)),
        pl.BlockSpec(kp8.shape, lambda i: (0, 0)),
        pl.BlockSpec(wf.shape, lambda i: (0, 0)),
    ]
    return in_specs


def _kpconv(g, nbx, xyz8, kp, w, tp):
    bn = g.shape[0]
    cin = g.shape[2]
    cout = w.shape[2]
    kp8 = jnp.pad(kp, ((0, 0), (0, 5)))
    wf = w.reshape(KS * cin, cout)
    return pl.pallas_call(
        _kpconv_kernel,
        grid=(bn // tp,),
        in_specs=_kpconv_specs(tp, cin, cout, kp8, wf),
        out_specs=[
            pl.BlockSpec((tp, cout), lambda i: (i, 0)),
            pl.BlockSpec((2, cout), lambda i: (0, 0)),
        ],
        out_shape=[
            jax.ShapeDtypeStruct((bn, cout), jnp.float32),
            jax.ShapeDtypeStruct((2, cout), jnp.float32),
        ],
    )(g, nbx, xyz8, kp8, wf)


def _kpconv_mask(g, nbx, xyz8, kp, w, tp):
    bn = g.shape[0]
    cin = g.shape[2]
    cout = w.shape[2]
    kp8 = jnp.pad(kp, ((0, 0), (0, 5)))
    wf = w.reshape(KS * cin, cout)
    return pl.pallas_call(
        _kpconv_mask_kernel,
        grid=(bn // tp,),
        in_specs=_kpconv_specs(tp, cin, cout, kp8, wf),
        out_specs=pl.BlockSpec((tp, 1), lambda i: (i, 0)),
        out_shape=jax.ShapeDtypeStruct((bn, 1), jnp.float32),
    )(g, nbx, xyz8, kp8, wf)


def _bn_scale_shift(sums_ref, g_ref, b_ref):
    s = sums_ref[...]
    mean = s[0:1, :] * (1.0 / BN)
    var = s[1:2, :] * (1.0 / BN) - mean * mean
    sc = g_ref[...] * lax.rsqrt(var + EPS)
    sh = b_ref[...] - mean * sc
    return sc, sh


def _bn_relu_kernel(raw_ref, sums_ref, g_ref, b_ref, out_ref):
    sc, sh = _bn_scale_shift(sums_ref, g_ref, b_ref)
    out_ref[...] = jnp.maximum(raw_ref[...] * sc + sh, 0.0)


def _bn_relu(raw, sums, g, b, tp):
    bn, c = raw.shape
    return pl.pallas_call(
        _bn_relu_kernel,
        grid=(bn // tp,),
        in_specs=[
            pl.BlockSpec((tp, c), lambda i: (i, 0)),
            pl.BlockSpec((2, c), lambda i: (0, 0)),
            pl.BlockSpec((1, c), lambda i: (0, 0)),
            pl.BlockSpec((1, c), lambda i: (0, 0)),
        ],
        out_specs=pl.BlockSpec((tp, c), lambda i: (i, 0)),
        out_shape=jax.ShapeDtypeStruct((bn, c), jnp.float32),
    )(raw, sums, g.reshape(1, c), b.reshape(1, c))


def _bn_relu_mask_kernel(raw_ref, sums_ref, g_ref, b_ref, chm_ref, ptm_ref,
                         out_ref):
    sc, sh = _bn_scale_shift(sums_ref, g_ref, b_ref)
    a = jnp.maximum(raw_ref[...] * sc + sh, 0.0)
    cm = chm_ref[...]
    w1 = jax.nn.sigmoid((cm[1:2, :] - cm[0:1, :]) * (1.0 / TAU))
    gate = ptm_ref[...] * w1 + (1.0 - w1)
    out_ref[...] = a * gate


def _bn_relu_mask(raw, sums, g, b, chml, ptm, tp):
    bn, c = raw.shape
    return pl.pallas_call(
        _bn_relu_mask_kernel,
        grid=(bn // tp,),
        in_specs=[
            pl.BlockSpec((tp, c), lambda i: (i, 0)),
            pl.BlockSpec((2, c), lambda i: (0, 0)),
            pl.BlockSpec((1, c), lambda i: (0, 0)),
            pl.BlockSpec((1, c), lambda i: (0, 0)),
            pl.BlockSpec((2, c), lambda i: (0, 0)),
            pl.BlockSpec((tp, 1), lambda i: (i, 0)),
        ],
        out_specs=pl.BlockSpec((tp, c), lambda i: (i, 0)),
        out_shape=jax.ShapeDtypeStruct((bn, c), jnp.float32),
    )(raw, sums, g.reshape(1, c), b.reshape(1, c), chml, ptm)


def _tail_mm_kernel(a_ref, b_ref, wa_ref, wb_ref, bias_ref, raw_ref, sums_ref):
    raw = (jnp.dot(a_ref[...], wa_ref[...], preferred_element_type=jnp.float32)
           + jnp.dot(b_ref[...], wb_ref[...], preferred_element_type=jnp.float32)
           + bias_ref[...])
    raw_ref[...] = raw
    _accum_stats(raw, sums_ref)


def _tail_mm(o0, o1, wa, wb, bias, tp):
    bn, c = o0.shape
    cout = wa.shape[1]
    return pl.pallas_call(
        _tail_mm_kernel,
        grid=(bn // tp,),
        in_specs=[
            pl.BlockSpec((tp, c), lambda i: (i, 0)),
            pl.BlockSpec((tp, c), lambda i: (i, 0)),
            pl.BlockSpec((c, cout), lambda i: (0, 0)),
            pl.BlockSpec((c, cout), lambda i: (0, 0)),
            pl.BlockSpec((1, cout), lambda i: (0, 0)),
        ],
        out_specs=[
            pl.BlockSpec((tp, cout), lambda i: (i, 0)),
            pl.BlockSpec((2, cout), lambda i: (0, 0)),
        ],
        out_shape=[
            jax.ShapeDtypeStruct((bn, cout), jnp.float32),
            jax.ShapeDtypeStruct((2, cout), jnp.float32),
        ],
    )(o0, o1, wa, wb, bias.reshape(1, cout))


def _tail_apply_kernel(raw_ref, sums_ref, g_ref, b_ref, fea_ref, out_ref):
    sc, sh = _bn_scale_shift(sums_ref, g_ref, b_ref)
    out_ref[...] = jnp.maximum(raw_ref[...] * sc + sh + fea_ref[...], 0.0)


def _tail_apply(raw, sums, g, b, fea, tp):
    bn, c = raw.shape
    return pl.pallas_call(
        _tail_apply_kernel,
        grid=(bn // tp,),
        in_specs=[
            pl.BlockSpec((tp, c), lambda i: (i, 0)),
            pl.BlockSpec((2, c), lambda i: (0, 0)),
            pl.BlockSpec((1, c), lambda i: (0, 0)),
            pl.BlockSpec((1, c), lambda i: (0, 0)),
            pl.BlockSpec((tp, c), lambda i: (i, 0)),
        ],
        out_specs=pl.BlockSpec((tp, c), lambda i: (i, 0)),
        out_shape=jax.ShapeDtypeStruct((bn, c), jnp.float32),
    )(raw, sums, g.reshape(1, c), b.reshape(1, c), fea)


def _flops_kernel(ptm_ref, chm_ref, out_ref):
    cm = chm_ref[...][0]
    w1 = jax.nn.sigmoid((cm[:, 1:2] - cm[:, 0:1]) * (1.0 / TAU))
    p = ptm_ref[...][0]
    lyr = pl.program_id(0)
    cin = lax.select(lyr == 0, jnp.float32(CIN), jnp.float32(COUT))
    scale = K_ * (cin + 1.0)
    res = (w1 * p + (1.0 - w1)) * scale
    out_ref[...] = res[None, None]


def _flops(ptm3, ch_mask):
    return pl.pallas_call(
        _flops_kernel,
        grid=(NL, B_),
        in_specs=[
            pl.BlockSpec((1, 1, N_), lambda l, b: (b, 0, 0)),
            pl.BlockSpec((1, COUT, 2), lambda l, b: (l, 0, 0)),
        ],
        out_specs=pl.BlockSpec((1, 1, COUT, N_), lambda l, b: (l, b, 0, 0)),
        out_shape=jax.ShapeDtypeStruct((NL, B_, COUT, N_), jnp.float32),
    )(ptm3, ch_mask)


def kernel(xyz, fea, knn_idx, ch_mask, body0_kp, body0_W, body0_g, body0_b,
           body1_kp, body1_W, body1_g, body1_b, ptm0_W, ptm0_g, ptm0_b,
           ptm1_kp, ptm1_W, ptm1_g, ptm1_b, ptm2_kp, ptm2_W, tail_W,
           tail_bias, tail_g, tail_bb):
    c4 = CIN // 4
    feaT = fea.transpose(0, 2, 1).reshape(BN, CIN)
    xyz8 = jnp.pad(xyz.transpose(0, 2, 1).reshape(BN, 3), ((0, 0), (0, 5)))
    off = (jnp.arange(B_, dtype=jnp.int32) * N_)[:, None, None]
    idxf = (knn_idx.astype(jnp.int32) + off).reshape(M_)

    nbx = _gather_rows(xyz8, idxf, 4096).reshape(BN, K_, 8)
    gfea = _gather_rows(feaT, idxf, 512).reshape(BN, K_, CIN)

    p0raw, p0sums = _mm_stats(feaT, ptm0_W, 512)
    p0a = _bn_relu(p0raw, p0sums, ptm0_g, ptm0_b, 512)
    g1 = _gather_rows(p0a, idxf, 2048).reshape(BN, K_, c4)
    p1raw, p1sums = _kpconv(g1, nbx, xyz8, ptm1_kp, ptm1_W, 256)
    p1a = _bn_relu(p1raw, p1sums, ptm1_g, ptm1_b, 512)
    g2 = _gather_rows(p1a, idxf, 2048).reshape(BN, K_, c4)
    ptm = _kpconv_mask(g2, nbx, xyz8, ptm2_kp, ptm2_W, 256)

    r0, s0 = _kpconv(gfea, nbx, xyz8, body0_kp, body0_W, 256)
    o0 = _bn_relu_mask(r0, s0, body0_g, body0_b, ch_mask[0].T, ptm, 512)
    g3 = _gather_rows(o0, idxf, 512).reshape(BN, K_, COUT)
    r1, s1 = _kpconv(g3, nbx, xyz8, body1_kp, body1_W, 256)
    o1 = _bn_relu_mask(r1, s1, body1_g, body1_b, ch_mask[1].T, ptm, 512)

    trw, tsums = _tail_mm(o0, o1, tail_W[:COUT], tail_W[COUT:], tail_bias, 512)
    outT = _tail_apply(trw, tsums, tail_g, tail_bb, feaT, 512)
    out = outT.reshape(B_, N_, COUT).transpose(0, 2, 1)

    flops = _flops(ptm.reshape(B_, 1, N_), ch_mask).reshape(-1)
    total = jnp.float32(sum(B_ * N_ * K_ * COUT * ((CIN if i == 0 else COUT) + 1)
                            for i in range(NL)))
    return (out, flops, total)


# speedup vs baseline: 12.8924x; 3.7235x over previous
import functools

import jax
import jax.numpy as jnp
from jax import lax
from jax.experimental import pallas as pl
from jax.experimental.pallas import tpu as pltpu
from jax.experimental.pallas import tpu_sc as plsc

B_, N_, K_, CIN, COUT, KS, NL = 2, 4096, 16, 128, 128, 5, 2
BN = B_ * N_
M_ = BN * K_
RADIUS, TAU, EPS = 1.0, 1.0, 1e-5


def _sc_workers():
    try:
        info = plsc.get_sparse_core_info()
        return info.num_cores, info.num_subcores
    except Exception:
        return 2, 16


def _gather_rows(table, idx, chunk):
    _, d = table.shape
    (m,) = idx.shape
    nc, ns = _sc_workers()
    per_w = m // (nc * ns)
    nch = per_w // chunk
    mesh = plsc.VectorSubcoreMesh(core_axis_name="c", subcore_axis_name="s")

    @functools.partial(
        pl.kernel,
        out_type=jax.ShapeDtypeStruct((m, d), jnp.float32),
        mesh=mesh,
        scratch_types=[
            pltpu.VMEM((chunk,), jnp.int32),
            pltpu.VMEM((chunk, d), jnp.float32),
            pltpu.SemaphoreType.DMA,
        ],
        compiler_params=pltpu.CompilerParams(use_tc_tiling_on_sc=False),
    )
    def gk(table_hbm, idx_hbm, out_hbm, idx_v, rows_v, sem):
        wid = lax.axis_index("s") * nc + lax.axis_index("c")
        base = wid * per_w

        def body(i, carry):
            off = base + i * chunk
            pltpu.sync_copy(idx_hbm.at[pl.ds(off, chunk)], idx_v)
            pltpu.async_copy(table_hbm.at[idx_v], rows_v, sem).wait()
            pltpu.sync_copy(rows_v, out_hbm.at[pl.ds(off, chunk)])
            return carry

        lax.fori_loop(0, nch, body, 0)

    return gk(table, idx)


def _accum_stats(raw, sums_ref):
    @pl.when(pl.program_id(0) == 0)
    def _():
        sums_ref[...] = jnp.zeros_like(sums_ref)

    s1 = jnp.sum(raw, axis=0, keepdims=True)
    s2 = jnp.sum(raw * raw, axis=0, keepdims=True)
    sums_ref[...] = sums_ref[...] + jnp.concatenate([s1, s2], axis=0)


def _mm_stats_kernel(x_ref, w_ref, raw_ref, sums_ref):
    raw = jnp.dot(x_ref[...], w_ref[...], preferred_element_type=jnp.float32)
    raw_ref[...] = raw
    _accum_stats(raw, sums_ref)


def _mm_stats(x, w, tp):
    bn, cin = x.shape
    cout = w.shape[1]
    return pl.pallas_call(
        _mm_stats_kernel,
        grid=(bn // tp,),
        in_specs=[
            pl.BlockSpec((tp, cin), lambda i: (i, 0)),
            pl.BlockSpec((cin, cout), lambda i: (0, 0)),
        ],
        out_specs=[
            pl.BlockSpec((tp, cout), lambda i: (i, 0)),
            pl.BlockSpec((2, cout), lambda i: (0, 0)),
        ],
        out_shape=[
            jax.ShapeDtypeStruct((bn, cout), jnp.float32),
            jax.ShapeDtypeStruct((2, cout), jnp.float32),
        ],
    )(x, w)


def _kpconv_core(g_ref, nbp_ref, ctr_ref, kpm_ref, smat_ref, kp2_ref, w_ref):
    cin = g_ref.shape[2]
    nbp = nbp_ref[...]
    ctr8 = ctr_ref[...]
    lane = lax.broadcasted_iota(jnp.int32, (8, K_ * 8), 1)
    sub = lax.broadcasted_iota(jnp.int32, (8, K_ * 8), 0)
    t8 = (lane % 8 == sub).astype(jnp.float32)
    ctrt = jnp.dot(ctr8, t8, preferred_element_type=jnp.float32)
    rel = (nbp - ctrt) * (1.0 / RADIUS)
    sq = rel * rel
    d2 = (jnp.dot(sq, smat_ref[...], preferred_element_type=jnp.float32)
          - 2.0 * jnp.dot(rel, kpm_ref[...], preferred_element_type=jnp.float32)
          + kp2_ref[...])
    infl = jnp.maximum(1.0 - jnp.sqrt(d2 + 1e-12), 0.0)
    outs = []
    for m in range(KS):
        repm = (lax.broadcasted_iota(jnp.int32, (8, cin), 0) == m
                ).astype(jnp.float32)
        acc = None
        for k in range(K_):
            bk = jnp.dot(infl[:, k * 8:(k + 1) * 8], repm,
                         preferred_element_type=jnp.float32)
            term = bk * g_ref[k]
            acc = term if acc is None else acc + term
        outs.append(acc)
    cat = jnp.concatenate(outs, axis=1)
    return jnp.dot(cat, w_ref[...], preferred_element_type=jnp.float32)


def _kpconv_kernel(g_ref, nb_ref, ctr_ref, kpm_ref, smat_ref, kp2_ref, w_ref,
                   raw_ref, sums_ref):
    raw = _kpconv_core(g_ref, nb_ref, ctr_ref, kpm_ref, smat_ref, kp2_ref,
                       w_ref)
    raw_ref[...] = raw
    _accum_stats(raw, sums_ref)


def _kpconv_mask_kernel(g_ref, nb_ref, ctr_ref, kpm_ref, smat_ref, kp2_ref,
                        w_ref, out_ref):
    raw = _kpconv_core(g_ref, nb_ref, ctr_ref, kpm_ref, smat_ref, kp2_ref,
                       w_ref)
    out_ref[...] = jax.nn.sigmoid((raw[:, 1:2] - raw[:, 0:1]) / TAU)


def _kp_prep(kp, w, cin, cout):
    kp8 = jnp.pad(kp, ((0, 0), (0, 5)))
    kpt8 = jnp.pad(kp8.T, ((0, 0), (0, 8 - KS)))
    eye = jnp.eye(K_, dtype=jnp.float32)
    kpm = jnp.kron(eye, kpt8)
    smat = jnp.kron(eye, jnp.ones((8, 8), jnp.float32))
    kp2 = jnp.tile(jnp.pad(jnp.sum(kp * kp, axis=1), (0, 8 - KS)), K_)[None]
    wf = w.reshape(KS * cin, cout)
    return kpm, smat, kp2, wf


def _kpconv_specs(tp, cin, wf):
    return [
        pl.BlockSpec((K_, tp, cin), lambda i: (0, i, 0)),
        pl.BlockSpec((tp, K_ * 8), lambda i: (i, 0)),
        pl.BlockSpec((tp, 8), lambda i: (i, 0)),
        pl.BlockSpec((K_ * 8, K_ * 8), lambda i: (0, 0)),
        pl.BlockSpec((K_ * 8, K_ * 8), lambda i: (0, 0)),
        pl.BlockSpec((1, K_ * 8), lambda i: (0, 0)),
        pl.BlockSpec(wf.shape, lambda i: (0, 0)),
    ]


def _kpconv(g, nbp, xyz8, kp, w, tp):
    cin = g.shape[2]
    cout = w.shape[2]
    bn = g.shape[1]
    kpm, smat, kp2, wf = _kp_prep(kp, w, cin, cout)
    return pl.pallas_call(
        _kpconv_kernel,
        grid=(bn // tp,),
        in_specs=_kpconv_specs(tp, cin, wf),
        out_specs=[
            pl.BlockSpec((tp, cout), lambda i: (i, 0)),
            pl.BlockSpec((2, cout), lambda i: (0, 0)),
        ],
        out_shape=[
            jax.ShapeDtypeStruct((bn, cout), jnp.float32),
            jax.ShapeDtypeStruct((2, cout), jnp.float32),
        ],
    )(g, nbp, xyz8, kpm, smat, kp2, wf)


def _kpconv_mask(g, nbp, xyz8, kp, w, tp):
    cin = g.shape[2]
    cout = w.shape[2]
    bn = g.shape[1]
    kpm, smat, kp2, wf = _kp_prep(kp, w, cin, cout)
    return pl.pallas_call(
        _kpconv_mask_kernel,
        grid=(bn // tp,),
        in_specs=_kpconv_specs(tp, cin, wf),
        out_specs=pl.BlockSpec((tp, 1), lambda i: (i, 0)),
        out_shape=jax.ShapeDtypeStruct((bn, 1), jnp.float32),
    )(g, nbp, xyz8, kpm, smat, kp2, wf)


def _bn_scale_shift(sums_ref, g_ref, b_ref):
    s = sums_ref[...]
    mean = s[0:1, :] * (1.0 / BN)
    var = s[1:2, :] * (1.0 / BN) - mean * mean
    sc = g_ref[...] * lax.rsqrt(var + EPS)
    sh = b_ref[...] - mean * sc
    return sc, sh


def _bn_relu_kernel(raw_ref, sums_ref, g_ref, b_ref, out_ref):
    sc, sh = _bn_scale_shift(sums_ref, g_ref, b_ref)
    out_ref[...] = jnp.maximum(raw_ref[...] * sc + sh, 0.0)


def _bn_relu(raw, sums, g, b, tp):
    bn, c = raw.shape
    return pl.pallas_call(
        _bn_relu_kernel,
        grid=(bn // tp,),
        in_specs=[
            pl.BlockSpec((tp, c), lambda i: (i, 0)),
            pl.BlockSpec((2, c), lambda i: (0, 0)),
            pl.BlockSpec((1, c), lambda i: (0, 0)),
            pl.BlockSpec((1, c), lambda i: (0, 0)),
        ],
        out_specs=pl.BlockSpec((tp, c), lambda i: (i, 0)),
        out_shape=jax.ShapeDtypeStruct((bn, c), jnp.float32),
    )(raw, sums, g.reshape(1, c), b.reshape(1, c))


def _bn_relu_mask_kernel(raw_ref, sums_ref, g_ref, b_ref, chm_ref, ptm_ref,
                         out_ref):
    sc, sh = _bn_scale_shift(sums_ref, g_ref, b_ref)
    a = jnp.maximum(raw_ref[...] * sc + sh, 0.0)
    cm = chm_ref[...]
    w1 = jax.nn.sigmoid((cm[1:2, :] - cm[0:1, :]) * (1.0 / TAU))
    gate = ptm_ref[...] * w1 + (1.0 - w1)
    out_ref[...] = a * gate


def _bn_relu_mask(raw, sums, g, b, chml, ptm, tp):
    bn, c = raw.shape
    return pl.pallas_call(
        _bn_relu_mask_kernel,
        grid=(bn // tp,),
        in_specs=[
            pl.BlockSpec((tp, c), lambda i: (i, 0)),
            pl.BlockSpec((2, c), lambda i: (0, 0)),
            pl.BlockSpec((1, c), lambda i: (0, 0)),
            pl.BlockSpec((1, c), lambda i: (0, 0)),
            pl.BlockSpec((2, c), lambda i: (0, 0)),
            pl.BlockSpec((tp, 1), lambda i: (i, 0)),
        ],
        out_specs=pl.BlockSpec((tp, c), lambda i: (i, 0)),
        out_shape=jax.ShapeDtypeStruct((bn, c), jnp.float32),
    )(raw, sums, g.reshape(1, c), b.reshape(1, c), chml, ptm)


def _tail_mm_kernel(a_ref, b_ref, wa_ref, wb_ref, bias_ref, raw_ref, sums_ref):
    raw = (jnp.dot(a_ref[...], wa_ref[...], preferred_element_type=jnp.float32)
           + jnp.dot(b_ref[...], wb_ref[...], preferred_element_type=jnp.float32)
           + bias_ref[...])
    raw_ref[...] = raw
    _accum_stats(raw, sums_ref)


def _tail_mm(o0, o1, wa, wb, bias, tp):
    bn, c = o0.shape
    cout = wa.shape[1]
    return pl.pallas_call(
        _tail_mm_kernel,
        grid=(bn // tp,),
        in_specs=[
            pl.BlockSpec((tp, c), lambda i: (i, 0)),
            pl.BlockSpec((tp, c), lambda i: (i, 0)),
            pl.BlockSpec((c, cout), lambda i: (0, 0)),
            pl.BlockSpec((c, cout), lambda i: (0, 0)),
            pl.BlockSpec((1, cout), lambda i: (0, 0)),
        ],
        out_specs=[
            pl.BlockSpec((tp, cout), lambda i: (i, 0)),
            pl.BlockSpec((2, cout), lambda i: (0, 0)),
        ],
        out_shape=[
            jax.ShapeDtypeStruct((bn, cout), jnp.float32),
            jax.ShapeDtypeStruct((2, cout), jnp.float32),
        ],
    )(o0, o1, wa, wb, bias.reshape(1, cout))


def _tail_apply_kernel(raw_ref, sums_ref, g_ref, b_ref, fea_ref, out_ref):
    sc, sh = _bn_scale_shift(sums_ref, g_ref, b_ref)
    out_ref[...] = jnp.maximum(raw_ref[...] * sc + sh + fea_ref[...], 0.0)


def _tail_apply(raw, sums, g, b, fea, tp):
    bn, c = raw.shape
    return pl.pallas_call(
        _tail_apply_kernel,
        grid=(bn // tp,),
        in_specs=[
            pl.BlockSpec((tp, c), lambda i: (i, 0)),
            pl.BlockSpec((2, c), lambda i: (0, 0)),
            pl.BlockSpec((1, c), lambda i: (0, 0)),
            pl.BlockSpec((1, c), lambda i: (0, 0)),
            pl.BlockSpec((tp, c), lambda i: (i, 0)),
        ],
        out_specs=pl.BlockSpec((tp, c), lambda i: (i, 0)),
        out_shape=jax.ShapeDtypeStruct((bn, c), jnp.float32),
    )(raw, sums, g.reshape(1, c), b.reshape(1, c), fea)


def _flops_kernel(ptm_ref, chm_ref, out_ref):
    cm = chm_ref[...][0]
    w1 = jax.nn.sigmoid((cm[:, 1:2] - cm[:, 0:1]) * (1.0 / TAU))
    p = ptm_ref[...][0]
    lyr = pl.program_id(0)
    cin = lax.select(lyr == 0, jnp.float32(CIN), jnp.float32(COUT))
    scale = K_ * (cin + 1.0)
    res = (w1 * p + (1.0 - w1)) * scale
    out_ref[...] = res[None, None]


def _flops(ptm3, ch_mask):
    return pl.pallas_call(
        _flops_kernel,
        grid=(NL, B_),
        in_specs=[
            pl.BlockSpec((1, 1, N_), lambda l, b: (b, 0, 0)),
            pl.BlockSpec((1, COUT, 2), lambda l, b: (l, 0, 0)),
        ],
        out_specs=pl.BlockSpec((1, 1, COUT, N_), lambda l, b: (l, b, 0, 0)),
        out_shape=jax.ShapeDtypeStruct((NL, B_, COUT, N_), jnp.float32),
    )(ptm3, ch_mask)


def kernel(xyz, fea, knn_idx, ch_mask, body0_kp, body0_W, body0_g, body0_b,
           body1_kp, body1_W, body1_g, body1_b, ptm0_W, ptm0_g, ptm0_b,
           ptm1_kp, ptm1_W, ptm1_g, ptm1_b, ptm2_kp, ptm2_W, tail_W,
           tail_bias, tail_g, tail_bb):
    c4 = CIN // 4
    feaT = fea.transpose(0, 2, 1).reshape(BN, CIN)
    xyz8 = jnp.pad(xyz.transpose(0, 2, 1).reshape(BN, 3), ((0, 0), (0, 5)))
    off = (jnp.arange(B_, dtype=jnp.int32) * N_)[:, None, None]
    idxg = knn_idx.astype(jnp.int32) + off
    idxf = idxg.reshape(M_)
    idxk = idxg.transpose(2, 0, 1).reshape(M_)

    nbp = _gather_rows(xyz8, idxf, 4096).reshape(BN, K_ * 8)
    gfea = _gather_rows(feaT, idxk, 512).reshape(K_, BN, CIN)

    p0raw, p0sums = _mm_stats(feaT, ptm0_W, 512)
    p0a = _bn_relu(p0raw, p0sums, ptm0_g, ptm0_b, 512)
    g1 = _gather_rows(p0a, idxk, 2048).reshape(K_, BN, c4)
    p1raw, p1sums = _kpconv(g1, nbp, xyz8, ptm1_kp, ptm1_W, 256)
    p1a = _bn_relu(p1raw, p1sums, ptm1_g, ptm1_b, 512)
    g2 = _gather_rows(p1a, idxk, 2048).reshape(K_, BN, c4)
    ptm = _kpconv_mask(g2, nbp, xyz8, ptm2_kp, ptm2_W, 256)

    r0, s0 = _kpconv(gfea, nbp, xyz8, body0_kp, body0_W, 256)
    o0 = _bn_relu_mask(r0, s0, body0_g, body0_b, ch_mask[0].T, ptm, 512)
    g3 = _gather_rows(o0, idxk, 512).reshape(K_, BN, COUT)
    r1, s1 = _kpconv(g3, nbp, xyz8, body1_kp, body1_W, 256)
    o1 = _bn_relu_mask(r1, s1, body1_g, body1_b, ch_mask[1].T, ptm, 512)

    trw, tsums = _tail_mm(o0, o1, tail_W[:COUT], tail_W[COUT:], tail_bias, 512)
    outT = _tail_apply(trw, tsums, tail_g, tail_bb, feaT, 512)
    out = outT.reshape(B_, N_, COUT).transpose(0, 2, 1)

    flops = _flops(ptm.reshape(B_, 1, N_), ch_mask).reshape(-1)
    total = jnp.float32(sum(B_ * N_ * K_ * COUT * ((CIN if i == 0 else COUT) + 1)
                            for i in range(NL)))
    return (out, flops, total)


# speedup vs baseline: 12.8927x; 1.0000x over previous
import functools

import jax
import jax.numpy as jnp
from jax import lax
from jax.experimental import pallas as pl
from jax.experimental.pallas import tpu as pltpu
from jax.experimental.pallas import tpu_sc as plsc

B_, N_, K_, CIN, COUT, KS, NL = 2, 4096, 16, 128, 128, 5, 2
BN = B_ * N_
M_ = BN * K_
RADIUS, TAU, EPS = 1.0, 1.0, 1e-5


def _sc_workers():
    try:
        info = plsc.get_sparse_core_info()
        return info.num_cores, info.num_subcores
    except Exception:
        return 2, 16


def _gather_rows(table, idx, chunk):
    _, d = table.shape
    (m,) = idx.shape
    nc, ns = _sc_workers()
    per_w = m // (nc * ns)
    nch = per_w // chunk
    mesh = plsc.VectorSubcoreMesh(core_axis_name="c", subcore_axis_name="s")

    @functools.partial(
        pl.kernel,
        out_type=jax.ShapeDtypeStruct((m, d), jnp.float32),
        mesh=mesh,
        scratch_types=[
            pltpu.VMEM((2, chunk), jnp.int32),
            pltpu.VMEM((2, chunk, d), jnp.float32),
            pltpu.SemaphoreType.DMA,
            pltpu.SemaphoreType.DMA,
            pltpu.SemaphoreType.DMA,
            pltpu.SemaphoreType.DMA,
        ],
        compiler_params=pltpu.CompilerParams(use_tc_tiling_on_sc=False),
    )
    def gk(table_hbm, idx_hbm, out_hbm, idx_v, rows_v, g0, g1, w0, w1):
        wid = lax.axis_index("s") * nc + lax.axis_index("c")
        base = wid * per_w
        gsem = (g0, g1)
        wsem = (w0, w1)
        gathers = [None, None]
        writes = [None, None]
        for i in range(nch):
            b = i & 1
            if writes[b] is not None:
                writes[b].wait()
            off = base + i * chunk
            pltpu.sync_copy(idx_hbm.at[pl.ds(off, chunk)], idx_v.at[b])
            gathers[b] = pltpu.async_copy(table_hbm.at[idx_v.at[b]],
                                          rows_v.at[b], gsem[b])
            pb = 1 - b
            if i >= 1 and gathers[pb] is not None:
                gathers[pb].wait()
                poff = base + (i - 1) * chunk
                writes[pb] = pltpu.async_copy(
                    rows_v.at[pb], out_hbm.at[pl.ds(poff, chunk)], wsem[pb])
        lb = (nch - 1) & 1
        gathers[lb].wait()
        loff = base + (nch - 1) * chunk
        writes[lb] = pltpu.async_copy(rows_v.at[lb],
                                      out_hbm.at[pl.ds(loff, chunk)], wsem[lb])
        for w in writes:
            if w is not None:
                w.wait()

    return gk(table, idx)


def _accum_stats(raw, sums_ref):
    @pl.when(pl.program_id(0) == 0)
    def _():
        sums_ref[...] = jnp.zeros_like(sums_ref)

    s1 = jnp.sum(raw, axis=0, keepdims=True)
    s2 = jnp.sum(raw * raw, axis=0, keepdims=True)
    sums_ref[...] = sums_ref[...] + jnp.concatenate([s1, s2], axis=0)


def _mm_stats_kernel(x_ref, w_ref, raw_ref, sums_ref):
    raw = jnp.dot(x_ref[...], w_ref[...], preferred_element_type=jnp.float32)
    raw_ref[...] = raw
    _accum_stats(raw, sums_ref)


def _mm_stats(x, w, tp):
    bn, cin = x.shape
    cout = w.shape[1]
    return pl.pallas_call(
        _mm_stats_kernel,
        grid=(bn // tp,),
        in_specs=[
            pl.BlockSpec((tp, cin), lambda i: (i, 0)),
            pl.BlockSpec((cin, cout), lambda i: (0, 0)),
        ],
        out_specs=[
            pl.BlockSpec((tp, cout), lambda i: (i, 0)),
            pl.BlockSpec((2, cout), lambda i: (0, 0)),
        ],
        out_shape=[
            jax.ShapeDtypeStruct((bn, cout), jnp.float32),
            jax.ShapeDtypeStruct((2, cout), jnp.float32),
        ],
    )(x, w)


def _kpconv_core(g_ref, nbp_ref, ctr_ref, kpm_ref, smat_ref, kp2_ref, w_ref):
    cin = g_ref.shape[2]
    nbp = nbp_ref[...]
    ctr8 = ctr_ref[...]
    lane = lax.broadcasted_iota(jnp.int32, (8, K_ * 8), 1)
    sub = lax.broadcasted_iota(jnp.int32, (8, K_ * 8), 0)
    t8 = (lane % 8 == sub).astype(jnp.float32)
    ctrt = jnp.dot(ctr8, t8, preferred_element_type=jnp.float32)
    rel = (nbp - ctrt) * (1.0 / RADIUS)
    sq = rel * rel
    d2 = (jnp.dot(sq, smat_ref[...], preferred_element_type=jnp.float32)
          - 2.0 * jnp.dot(rel, kpm_ref[...], preferred_element_type=jnp.float32)
          + kp2_ref[...])
    infl = jnp.maximum(1.0 - jnp.sqrt(d2 + 1e-12), 0.0)
    outs = []
    for m in range(KS):
        repm = (lax.broadcasted_iota(jnp.int32, (8, cin), 0) == m
                ).astype(jnp.float32)
        acc = None
        for k in range(K_):
            bk = jnp.dot(infl[:, k * 8:(k + 1) * 8], repm,
                         preferred_element_type=jnp.float32)
            term = bk * g_ref[k]
            acc = term if acc is None else acc + term
        outs.append(acc)
    cat = jnp.concatenate(outs, axis=1)
    return jnp.dot(cat, w_ref[...], preferred_element_type=jnp.float32)


def _kpconv_kernel(g_ref, nb_ref, ctr_ref, kpm_ref, smat_ref, kp2_ref, w_ref,
                   raw_ref, sums_ref):
    raw = _kpconv_core(g_ref, nb_ref, ctr_ref, kpm_ref, smat_ref, kp2_ref,
                       w_ref)
    raw_ref[...] = raw
    _accum_stats(raw, sums_ref)


def _kpconv_mask_kernel(g_ref, nb_ref, ctr_ref, kpm_ref, smat_ref, kp2_ref,
                        w_ref, out_ref):
    raw = _kpconv_core(g_ref, nb_ref, ctr_ref, kpm_ref, smat_ref, kp2_ref,
                       w_ref)
    out_ref[...] = jax.nn.sigmoid((raw[:, 1:2] - raw[:, 0:1]) / TAU)


def _kp_prep(kp, w, cin, cout):
    kp8 = jnp.pad(kp, ((0, 0), (0, 5)))
    kpt8 = jnp.pad(kp8.T, ((0, 0), (0, 8 - KS)))
    eye = jnp.eye(K_, dtype=jnp.float32)
    kpm = jnp.kron(eye, kpt8)
    smat = jnp.kron(eye, jnp.ones((8, 8), jnp.float32))
    kp2 = jnp.tile(jnp.pad(jnp.sum(kp * kp, axis=1), (0, 8 - KS)), K_)[None]
    wf = w.reshape(KS * cin, cout)
    return kpm, smat, kp2, wf


def _kpconv_specs(tp, cin, wf):
    return [
        pl.BlockSpec((K_, tp, cin), lambda i: (0, i, 0)),
        pl.BlockSpec((tp, K_ * 8), lambda i: (i, 0)),
        pl.BlockSpec((tp, 8), lambda i: (i, 0)),
        pl.BlockSpec((K_ * 8, K_ * 8), lambda i: (0, 0)),
        pl.BlockSpec((K_ * 8, K_ * 8), lambda i: (0, 0)),
        pl.BlockSpec((1, K_ * 8), lambda i: (0, 0)),
        pl.BlockSpec(wf.shape, lambda i: (0, 0)),
    ]


def _kpconv(g, nbp, xyz8, kp, w, tp):
    cin = g.shape[2]
    cout = w.shape[2]
    bn = g.shape[1]
    kpm, smat, kp2, wf = _kp_prep(kp, w, cin, cout)
    return pl.pallas_call(
        _kpconv_kernel,
        grid=(bn // tp,),
        in_specs=_kpconv_specs(tp, cin, wf),
        out_specs=[
            pl.BlockSpec((tp, cout), lambda i: (i, 0)),
            pl.BlockSpec((2, cout), lambda i: (0, 0)),
        ],
        out_shape=[
            jax.ShapeDtypeStruct((bn, cout), jnp.float32),
            jax.ShapeDtypeStruct((2, cout), jnp.float32),
        ],
    )(g, nbp, xyz8, kpm, smat, kp2, wf)


def _kpconv_mask(g, nbp, xyz8, kp, w, tp):
    cin = g.shape[2]
    cout = w.shape[2]
    bn = g.shape[1]
    kpm, smat, kp2, wf = _kp_prep(kp, w, cin, cout)
    return pl.pallas_call(
        _kpconv_mask_kernel,
        grid=(bn // tp,),
        in_specs=_kpconv_specs(tp, cin, wf),
        out_specs=pl.BlockSpec((tp, 1), lambda i: (i, 0)),
        out_shape=jax.ShapeDtypeStruct((bn, 1), jnp.float32),
    )(g, nbp, xyz8, kpm, smat, kp2, wf)


def _bn_scale_shift(sums_ref, g_ref, b_ref):
    s = sums_ref[...]
    mean = s[0:1, :] * (1.0 / BN)
    var = s[1:2, :] * (1.0 / BN) - mean * mean
    sc = g_ref[...] * lax.rsqrt(var + EPS)
    sh = b_ref[...] - mean * sc
    return sc, sh


def _bn_relu_kernel(raw_ref, sums_ref, g_ref, b_ref, out_ref):
    sc, sh = _bn_scale_shift(sums_ref, g_ref, b_ref)
    out_ref[...] = jnp.maximum(raw_ref[...] * sc + sh, 0.0)


def _bn_relu(raw, sums, g, b, tp):
    bn, c = raw.shape
    return pl.pallas_call(
        _bn_relu_kernel,
        grid=(bn // tp,),
        in_specs=[
            pl.BlockSpec((tp, c), lambda i: (i, 0)),
            pl.BlockSpec((2, c), lambda i: (0, 0)),
            pl.BlockSpec((1, c), lambda i: (0, 0)),
            pl.BlockSpec((1, c), lambda i: (0, 0)),
        ],
        out_specs=pl.BlockSpec((tp, c), lambda i: (i, 0)),
        out_shape=jax.ShapeDtypeStruct((bn, c), jnp.float32),
    )(raw, sums, g.reshape(1, c), b.reshape(1, c))


def _bn_relu_mask_kernel(raw_ref, sums_ref, g_ref, b_ref, chm_ref, ptm_ref,
                         out_ref):
    sc, sh = _bn_scale_shift(sums_ref, g_ref, b_ref)
    a = jnp.maximum(raw_ref[...] * sc + sh, 0.0)
    cm = chm_ref[...]
    w1 = jax.nn.sigmoid((cm[1:2, :] - cm[0:1, :]) * (1.0 / TAU))
    gate = ptm_ref[...] * w1 + (1.0 - w1)
    out_ref[...] = a * gate


def _bn_relu_mask(raw, sums, g, b, chml, ptm, tp):
    bn, c = raw.shape
    return pl.pallas_call(
        _bn_relu_mask_kernel,
        grid=(bn // tp,),
        in_specs=[
            pl.BlockSpec((tp, c), lambda i: (i, 0)),
            pl.BlockSpec((2, c), lambda i: (0, 0)),
            pl.BlockSpec((1, c), lambda i: (0, 0)),
            pl.BlockSpec((1, c), lambda i: (0, 0)),
            pl.BlockSpec((2, c), lambda i: (0, 0)),
            pl.BlockSpec((tp, 1), lambda i: (i, 0)),
        ],
        out_specs=pl.BlockSpec((tp, c), lambda i: (i, 0)),
        out_shape=jax.ShapeDtypeStruct((bn, c), jnp.float32),
    )(raw, sums, g.reshape(1, c), b.reshape(1, c), chml, ptm)


def _tail_mm_kernel(a_ref, b_ref, wa_ref, wb_ref, bias_ref, raw_ref, sums_ref):
    raw = (jnp.dot(a_ref[...], wa_ref[...], preferred_element_type=jnp.float32)
           + jnp.dot(b_ref[...], wb_ref[...], preferred_element_type=jnp.float32)
           + bias_ref[...])
    raw_ref[...] = raw
    _accum_stats(raw, sums_ref)


def _tail_mm(o0, o1, wa, wb, bias, tp):
    bn, c = o0.shape
    cout = wa.shape[1]
    return pl.pallas_call(
        _tail_mm_kernel,
        grid=(bn // tp,),
        in_specs=[
            pl.BlockSpec((tp, c), lambda i: (i, 0)),
            pl.BlockSpec((tp, c), lambda i: (i, 0)),
            pl.BlockSpec((c, cout), lambda i: (0, 0)),
            pl.BlockSpec((c, cout), lambda i: (0, 0)),
            pl.BlockSpec((1, cout), lambda i: (0, 0)),
        ],
        out_specs=[
            pl.BlockSpec((tp, cout), lambda i: (i, 0)),
            pl.BlockSpec((2, cout), lambda i: (0, 0)),
        ],
        out_shape=[
            jax.ShapeDtypeStruct((bn, cout), jnp.float32),
            jax.ShapeDtypeStruct((2, cout), jnp.float32),
        ],
    )(o0, o1, wa, wb, bias.reshape(1, cout))


def _tail_apply_kernel(raw_ref, sums_ref, g_ref, b_ref, fea_ref, out_ref):
    sc, sh = _bn_scale_shift(sums_ref, g_ref, b_ref)
    out_ref[...] = jnp.maximum(raw_ref[...] * sc + sh + fea_ref[...], 0.0)


def _tail_apply(raw, sums, g, b, fea, tp):
    bn, c = raw.shape
    return pl.pallas_call(
        _tail_apply_kernel,
        grid=(bn // tp,),
        in_specs=[
            pl.BlockSpec((tp, c), lambda i: (i, 0)),
            pl.BlockSpec((2, c), lambda i: (0, 0)),
            pl.BlockSpec((1, c), lambda i: (0, 0)),
            pl.BlockSpec((1, c), lambda i: (0, 0)),
            pl.BlockSpec((tp, c), lambda i: (i, 0)),
        ],
        out_specs=pl.BlockSpec((tp, c), lambda i: (i, 0)),
        out_shape=jax.ShapeDtypeStruct((bn, c), jnp.float32),
    )(raw, sums, g.reshape(1, c), b.reshape(1, c), fea)


def _flops_kernel(ptm_ref, chm_ref, out_ref):
    cm = chm_ref[...][0]
    w1 = jax.nn.sigmoid((cm[:, 1:2] - cm[:, 0:1]) * (1.0 / TAU))
    p = ptm_ref[...][0]
    lyr = pl.program_id(0)
    cin = lax.select(lyr == 0, jnp.float32(CIN), jnp.float32(COUT))
    scale = K_ * (cin + 1.0)
    res = (w1 * p + (1.0 - w1)) * scale
    out_ref[...] = res[None, None]


def _flops(ptm3, ch_mask):
    return pl.pallas_call(
        _flops_kernel,
        grid=(NL, B_),
        in_specs=[
            pl.BlockSpec((1, 1, N_), lambda l, b: (b, 0, 0)),
            pl.BlockSpec((1, COUT, 2), lambda l, b: (l, 0, 0)),
        ],
        out_specs=pl.BlockSpec((1, 1, COUT, N_), lambda l, b: (l, b, 0, 0)),
        out_shape=jax.ShapeDtypeStruct((NL, B_, COUT, N_), jnp.float32),
    )(ptm3, ch_mask)


def kernel(xyz, fea, knn_idx, ch_mask, body0_kp, body0_W, body0_g, body0_b,
           body1_kp, body1_W, body1_g, body1_b, ptm0_W, ptm0_g, ptm0_b,
           ptm1_kp, ptm1_W, ptm1_g, ptm1_b, ptm2_kp, ptm2_W, tail_W,
           tail_bias, tail_g, tail_bb):
    c4 = CIN // 4
    feaT = fea.transpose(0, 2, 1).reshape(BN, CIN)
    xyz8 = jnp.pad(xyz.transpose(0, 2, 1).reshape(BN, 3), ((0, 0), (0, 5)))
    off = (jnp.arange(B_, dtype=jnp.int32) * N_)[:, None, None]
    idxg = knn_idx.astype(jnp.int32) + off
    idxf = idxg.reshape(M_)
    idxk = idxg.transpose(2, 0, 1).reshape(M_)

    nbp = _gather_rows(xyz8, idxf, 1024).reshape(BN, K_ * 8)
    gfea = _gather_rows(feaT, idxk, 256).reshape(K_, BN, CIN)

    p0raw, p0sums = _mm_stats(feaT, ptm0_W, 512)
    p0a = _bn_relu(p0raw, p0sums, ptm0_g, ptm0_b, 512)
    g1 = _gather_rows(p0a, idxk, 1024).reshape(K_, BN, c4)
    p1raw, p1sums = _kpconv(g1, nbp, xyz8, ptm1_kp, ptm1_W, 256)
    p1a = _bn_relu(p1raw, p1sums, ptm1_g, ptm1_b, 512)
    g2 = _gather_rows(p1a, idxk, 1024).reshape(K_, BN, c4)
    ptm = _kpconv_mask(g2, nbp, xyz8, ptm2_kp, ptm2_W, 256)

    r0, s0 = _kpconv(gfea, nbp, xyz8, body0_kp, body0_W, 256)
    o0 = _bn_relu_mask(r0, s0, body0_g, body0_b, ch_mask[0].T, ptm, 512)
    g3 = _gather_rows(o0, idxk, 256).reshape(K_, BN, COUT)
    r1, s1 = _kpconv(g3, nbp, xyz8, body1_kp, body1_W, 256)
    o1 = _bn_relu_mask(r1, s1, body1_g, body1_b, ch_mask[1].T, ptm, 512)

    trw, tsums = _tail_mm(o0, o1, tail_W[:COUT], tail_W[COUT:], tail_bias, 512)
    outT = _tail_apply(trw, tsums, tail_g, tail_bb, feaT, 512)
    out = outT.reshape(B_, N_, COUT).transpose(0, 2, 1)

    flops = _flops(ptm.reshape(B_, 1, N_), ch_mask).reshape(-1)
    total = jnp.float32(sum(B_ * N_ * K_ * COUT * ((CIN if i == 0 else COUT) + 1)
                            for i in range(NL)))
    return (out, flops, total)


# speedup vs baseline: 13.5553x; 1.0514x over previous
import functools

import jax
import jax.numpy as jnp
from jax import lax
from jax.experimental import pallas as pl
from jax.experimental.pallas import tpu as pltpu
from jax.experimental.pallas import tpu_sc as plsc

B_, N_, K_, CIN, COUT, KS, NL = 2, 4096, 16, 128, 128, 5, 2
BN = B_ * N_
M_ = BN * K_
RADIUS, TAU, EPS = 1.0, 1.0, 1e-5


def _sc_workers():
    try:
        info = plsc.get_sparse_core_info()
        return info.num_cores, info.num_subcores
    except Exception:
        return 2, 16


def _gather_rows(table, idx, chunk):
    _, d = table.shape
    (m,) = idx.shape
    nc, ns = _sc_workers()
    per_w = m // (nc * ns)
    nch = per_w // chunk
    mesh = plsc.VectorSubcoreMesh(core_axis_name="c", subcore_axis_name="s")

    @functools.partial(
        pl.kernel,
        out_type=jax.ShapeDtypeStruct((m, d), jnp.float32),
        mesh=mesh,
        scratch_types=[
            pltpu.VMEM((2, chunk), jnp.int32),
            pltpu.VMEM((2, chunk, d), jnp.float32),
            pltpu.SemaphoreType.DMA,
            pltpu.SemaphoreType.DMA,
            pltpu.SemaphoreType.DMA,
            pltpu.SemaphoreType.DMA,
        ],
        compiler_params=pltpu.CompilerParams(use_tc_tiling_on_sc=False),
    )
    def gk(table_hbm, idx_hbm, out_hbm, idx_v, rows_v, g0, g1, w0, w1):
        wid = lax.axis_index("s") * nc + lax.axis_index("c")
        base = wid * per_w
        gsem = (g0, g1)
        wsem = (w0, w1)
        gathers = [None, None]
        writes = [None, None]
        for i in range(nch):
            b = i & 1
            if writes[b] is not None:
                writes[b].wait()
            off = base + i * chunk
            pltpu.sync_copy(idx_hbm.at[pl.ds(off, chunk)], idx_v.at[b])
            gathers[b] = pltpu.async_copy(table_hbm.at[idx_v.at[b]],
                                          rows_v.at[b], gsem[b])
            pb = 1 - b
            if i >= 1 and gathers[pb] is not None:
                gathers[pb].wait()
                poff = base + (i - 1) * chunk
                writes[pb] = pltpu.async_copy(
                    rows_v.at[pb], out_hbm.at[pl.ds(poff, chunk)], wsem[pb])
        lb = (nch - 1) & 1
        gathers[lb].wait()
        loff = base + (nch - 1) * chunk
        writes[lb] = pltpu.async_copy(rows_v.at[lb],
                                      out_hbm.at[pl.ds(loff, chunk)], wsem[lb])
        for w in writes:
            if w is not None:
                w.wait()

    return gk(table, idx)


def _accum_stats(raw, sums_ref):
    @pl.when(pl.program_id(0) == 0)
    def _():
        sums_ref[...] = jnp.zeros_like(sums_ref)

    s1 = jnp.sum(raw, axis=0, keepdims=True)
    s2 = jnp.sum(raw * raw, axis=0, keepdims=True)
    sums_ref[...] = sums_ref[...] + jnp.concatenate([s1, s2], axis=0)


def _mm_stats_kernel(x_ref, w_ref, raw_ref, sums_ref):
    raw = jnp.dot(x_ref[...], w_ref[...], preferred_element_type=jnp.float32)
    raw_ref[...] = raw
    _accum_stats(raw, sums_ref)


def _mm_stats(x, w, tp):
    bn, cin = x.shape
    cout = w.shape[1]
    return pl.pallas_call(
        _mm_stats_kernel,
        grid=(bn // tp,),
        in_specs=[
            pl.BlockSpec((tp, cin), lambda i: (i, 0)),
            pl.BlockSpec((cin, cout), lambda i: (0, 0)),
        ],
        out_specs=[
            pl.BlockSpec((tp, cout), lambda i: (i, 0)),
            pl.BlockSpec((2, cout), lambda i: (0, 0)),
        ],
        out_shape=[
            jax.ShapeDtypeStruct((bn, cout), jnp.float32),
            jax.ShapeDtypeStruct((2, cout), jnp.float32),
        ],
    )(x, w)


def _kpconv_core(g_ref, nbp_ref, ctr_ref, kpm_ref, smat_ref, kp2_ref, w_ref):
    cin = g_ref.shape[2]
    nbp = nbp_ref[...]
    ctr8 = ctr_ref[...]
    lane = lax.broadcasted_iota(jnp.int32, (8, K_ * 8), 1)
    sub = lax.broadcasted_iota(jnp.int32, (8, K_ * 8), 0)
    t8 = (lane % 8 == sub).astype(jnp.float32)
    ctrt = jnp.dot(ctr8, t8, preferred_element_type=jnp.float32)
    rel = (nbp - ctrt) * (1.0 / RADIUS)
    sq = rel * rel
    d2 = (jnp.dot(sq, smat_ref[...], preferred_element_type=jnp.float32)
          - 2.0 * jnp.dot(rel, kpm_ref[...], preferred_element_type=jnp.float32)
          + kp2_ref[...])
    infl = jnp.maximum(1.0 - jnp.sqrt(d2 + 1e-12), 0.0)
    outs = []
    for m in range(KS):
        repm = (lax.broadcasted_iota(jnp.int32, (8, cin), 0) == m
                ).astype(jnp.float32)
        acc = None
        for k in range(K_):
            bk = jnp.dot(infl[:, k * 8:(k + 1) * 8], repm,
                         preferred_element_type=jnp.float32)
            term = bk * g_ref[k]
            acc = term if acc is None else acc + term
        outs.append(acc)
    cat = jnp.concatenate(outs, axis=1)
    return jnp.dot(cat, w_ref[...], preferred_element_type=jnp.float32)


def _kpconv_kernel(g_ref, nb_ref, ctr_ref, kpm_ref, smat_ref, kp2_ref, w_ref,
                   raw_ref, sums_ref):
    raw = _kpconv_core(g_ref, nb_ref, ctr_ref, kpm_ref, smat_ref, kp2_ref,
                       w_ref)
    raw_ref[...] = raw
    _accum_stats(raw, sums_ref)


def _kpconv_mask_kernel(g_ref, nb_ref, ctr_ref, kpm_ref, smat_ref, kp2_ref,
                        w_ref, out_ref):
    raw = _kpconv_core(g_ref, nb_ref, ctr_ref, kpm_ref, smat_ref, kp2_ref,
                       w_ref)
    out_ref[...] = jax.nn.sigmoid((raw[:, 1:2] - raw[:, 0:1]) / TAU)


def _kp_prep(kp, w, cin, cout):
    kp8 = jnp.pad(kp, ((0, 0), (0, 5)))
    kpt8 = jnp.pad(kp8.T, ((0, 0), (0, 8 - KS)))
    eye = jnp.eye(K_, dtype=jnp.float32)
    kpm = jnp.kron(eye, kpt8)
    smat = jnp.kron(eye, jnp.ones((8, 8), jnp.float32))
    kp2 = jnp.tile(jnp.pad(jnp.sum(kp * kp, axis=1), (0, 8 - KS)), K_)[None]
    wf = w.reshape(KS * cin, cout)
    return kpm, smat, kp2, wf


def _kpconv_specs(tp, cin, wf):
    return [
        pl.BlockSpec((K_, tp, cin), lambda i: (0, i, 0)),
        pl.BlockSpec((tp, K_ * 8), lambda i: (i, 0)),
        pl.BlockSpec((tp, 8), lambda i: (i, 0)),
        pl.BlockSpec((K_ * 8, K_ * 8), lambda i: (0, 0)),
        pl.BlockSpec((K_ * 8, K_ * 8), lambda i: (0, 0)),
        pl.BlockSpec((1, K_ * 8), lambda i: (0, 0)),
        pl.BlockSpec(wf.shape, lambda i: (0, 0)),
    ]


def _kpconv(g, nbp, xyz8, kp, w, tp):
    cin = g.shape[2]
    cout = w.shape[2]
    bn = g.shape[1]
    kpm, smat, kp2, wf = _kp_prep(kp, w, cin, cout)
    return pl.pallas_call(
        _kpconv_kernel,
        grid=(bn // tp,),
        in_specs=_kpconv_specs(tp, cin, wf),
        out_specs=[
            pl.BlockSpec((tp, cout), lambda i: (i, 0)),
            pl.BlockSpec((2, cout), lambda i: (0, 0)),
        ],
        out_shape=[
            jax.ShapeDtypeStruct((bn, cout), jnp.float32),
            jax.ShapeDtypeStruct((2, cout), jnp.float32),
        ],
    )(g, nbp, xyz8, kpm, smat, kp2, wf)


def _kpconv_mask(g, nbp, xyz8, kp, w, tp):
    cin = g.shape[2]
    cout = w.shape[2]
    bn = g.shape[1]
    kpm, smat, kp2, wf = _kp_prep(kp, w, cin, cout)
    return pl.pallas_call(
        _kpconv_mask_kernel,
        grid=(bn // tp,),
        in_specs=_kpconv_specs(tp, cin, wf),
        out_specs=pl.BlockSpec((tp, 1), lambda i: (i, 0)),
        out_shape=jax.ShapeDtypeStruct((bn, 1), jnp.float32),
    )(g, nbp, xyz8, kpm, smat, kp2, wf)


def _bn_scale_shift(sums_ref, g_ref, b_ref):
    s = sums_ref[...]
    mean = s[0:1, :] * (1.0 / BN)
    var = s[1:2, :] * (1.0 / BN) - mean * mean
    sc = g_ref[...] * lax.rsqrt(var + EPS)
    sh = b_ref[...] - mean * sc
    return sc, sh


def _bn_relu_kernel(raw_ref, sums_ref, g_ref, b_ref, out_ref):
    sc, sh = _bn_scale_shift(sums_ref, g_ref, b_ref)
    out_ref[...] = jnp.maximum(raw_ref[...] * sc + sh, 0.0)


def _bn_relu(raw, sums, g, b, tp):
    bn, c = raw.shape
    return pl.pallas_call(
        _bn_relu_kernel,
        grid=(bn // tp,),
        in_specs=[
            pl.BlockSpec((tp, c), lambda i: (i, 0)),
            pl.BlockSpec((2, c), lambda i: (0, 0)),
            pl.BlockSpec((1, c), lambda i: (0, 0)),
            pl.BlockSpec((1, c), lambda i: (0, 0)),
        ],
        out_specs=pl.BlockSpec((tp, c), lambda i: (i, 0)),
        out_shape=jax.ShapeDtypeStruct((bn, c), jnp.float32),
    )(raw, sums, g.reshape(1, c), b.reshape(1, c))


def _bn_relu_mask_kernel(raw_ref, sums_ref, g_ref, b_ref, chm_ref, ptm_ref,
                         out_ref):
    sc, sh = _bn_scale_shift(sums_ref, g_ref, b_ref)
    a = jnp.maximum(raw_ref[...] * sc + sh, 0.0)
    cm = chm_ref[...]
    w1 = jax.nn.sigmoid((cm[1:2, :] - cm[0:1, :]) * (1.0 / TAU))
    gate = ptm_ref[...] * w1 + (1.0 - w1)
    out_ref[...] = a * gate


def _bn_relu_mask(raw, sums, g, b, chml, ptm, tp):
    bn, c = raw.shape
    return pl.pallas_call(
        _bn_relu_mask_kernel,
        grid=(bn // tp,),
        in_specs=[
            pl.BlockSpec((tp, c), lambda i: (i, 0)),
            pl.BlockSpec((2, c), lambda i: (0, 0)),
            pl.BlockSpec((1, c), lambda i: (0, 0)),
            pl.BlockSpec((1, c), lambda i: (0, 0)),
            pl.BlockSpec((2, c), lambda i: (0, 0)),
            pl.BlockSpec((tp, 1), lambda i: (i, 0)),
        ],
        out_specs=pl.BlockSpec((tp, c), lambda i: (i, 0)),
        out_shape=jax.ShapeDtypeStruct((bn, c), jnp.float32),
    )(raw, sums, g.reshape(1, c), b.reshape(1, c), chml, ptm)


def _tail_mm_kernel(a_ref, b_ref, wa_ref, wb_ref, bias_ref, raw_ref, sums_ref):
    raw = (jnp.dot(a_ref[...], wa_ref[...], preferred_element_type=jnp.float32)
           + jnp.dot(b_ref[...], wb_ref[...], preferred_element_type=jnp.float32)
           + bias_ref[...])
    raw_ref[...] = raw
    _accum_stats(raw, sums_ref)


def _tail_mm(o0, o1, wa, wb, bias, tp):
    bn, c = o0.shape
    cout = wa.shape[1]
    return pl.pallas_call(
        _tail_mm_kernel,
        grid=(bn // tp,),
        in_specs=[
            pl.BlockSpec((tp, c), lambda i: (i, 0)),
            pl.BlockSpec((tp, c), lambda i: (i, 0)),
            pl.BlockSpec((c, cout), lambda i: (0, 0)),
            pl.BlockSpec((c, cout), lambda i: (0, 0)),
            pl.BlockSpec((1, cout), lambda i: (0, 0)),
        ],
        out_specs=[
            pl.BlockSpec((tp, cout), lambda i: (i, 0)),
            pl.BlockSpec((2, cout), lambda i: (0, 0)),
        ],
        out_shape=[
            jax.ShapeDtypeStruct((bn, cout), jnp.float32),
            jax.ShapeDtypeStruct((2, cout), jnp.float32),
        ],
    )(o0, o1, wa, wb, bias.reshape(1, cout))


def _tail_apply_kernel(raw_ref, sums_ref, g_ref, b_ref, fea_ref, out_ref):
    sc, sh = _bn_scale_shift(sums_ref, g_ref, b_ref)
    y = jnp.maximum(raw_ref[...] * sc + sh + fea_ref[...], 0.0)
    out_ref[...] = y.T[None]


def _tail_apply(raw, sums, g, b, fea, tp):
    bn, c = raw.shape
    npt = N_ // tp
    return pl.pallas_call(
        _tail_apply_kernel,
        grid=(bn // tp,),
        in_specs=[
            pl.BlockSpec((tp, c), lambda i: (i, 0)),
            pl.BlockSpec((2, c), lambda i: (0, 0)),
            pl.BlockSpec((1, c), lambda i: (0, 0)),
            pl.BlockSpec((1, c), lambda i: (0, 0)),
            pl.BlockSpec((tp, c), lambda i: (i, 0)),
        ],
        out_specs=pl.BlockSpec((1, c, tp), lambda i: (i // npt, 0, i % npt)),
        out_shape=jax.ShapeDtypeStruct((B_, c, N_), jnp.float32),
    )(raw, sums, g.reshape(1, c), b.reshape(1, c), fea)


def _to_rows_kernel(x_ref, out_ref):
    out_ref[...] = x_ref[0].T


def _to_rows(x, tp):
    b, c, n = x.shape
    npt = n // tp
    return pl.pallas_call(
        _to_rows_kernel,
        grid=(b * npt,),
        in_specs=[pl.BlockSpec((1, c, tp), lambda i: (i // npt, 0, i % npt))],
        out_specs=pl.BlockSpec((tp, c), lambda i: (i, 0)),
        out_shape=jax.ShapeDtypeStruct((b * n, c), jnp.float32),
    )(x)


def _flops_kernel(ptm_ref, chm_ref, out_ref):
    cm = chm_ref[...][0]
    w1 = jax.nn.sigmoid((cm[:, 1:2] - cm[:, 0:1]) * (1.0 / TAU))
    p = ptm_ref[...][0]
    lyr = pl.program_id(0)
    cin = lax.select(lyr == 0, jnp.float32(CIN), jnp.float32(COUT))
    scale = K_ * (cin + 1.0)
    res = (w1 * p + (1.0 - w1)) * scale
    out_ref[...] = res[None, None]


def _flops(ptm3, ch_mask):
    return pl.pallas_call(
        _flops_kernel,
        grid=(NL, B_),
        in_specs=[
            pl.BlockSpec((1, 1, N_), lambda l, b: (b, 0, 0)),
            pl.BlockSpec((1, COUT, 2), lambda l, b: (l, 0, 0)),
        ],
        out_specs=pl.BlockSpec((1, 1, COUT, N_), lambda l, b: (l, b, 0, 0)),
        out_shape=jax.ShapeDtypeStruct((NL, B_, COUT, N_), jnp.float32),
    )(ptm3, ch_mask)


def kernel(xyz, fea, knn_idx, ch_mask, body0_kp, body0_W, body0_g, body0_b,
           body1_kp, body1_W, body1_g, body1_b, ptm0_W, ptm0_g, ptm0_b,
           ptm1_kp, ptm1_W, ptm1_g, ptm1_b, ptm2_kp, ptm2_W, tail_W,
           tail_bias, tail_g, tail_bb):
    c4 = CIN // 4
    feaT = _to_rows(fea, 512)
    xyz8 = jnp.pad(xyz.transpose(0, 2, 1).reshape(BN, 3), ((0, 0), (0, 5)))
    off = (jnp.arange(B_, dtype=jnp.int32) * N_)[:, None, None]
    idxg = knn_idx.astype(jnp.int32) + off
    idxf = idxg.reshape(M_)
    idxk = idxg.transpose(2, 0, 1).reshape(M_)

    nbp = _gather_rows(xyz8, idxf, 1024).reshape(BN, K_ * 8)
    gfea = _gather_rows(feaT, idxk, 256).reshape(K_, BN, CIN)

    p0raw, p0sums = _mm_stats(feaT, ptm0_W, 512)
    p0a = _bn_relu(p0raw, p0sums, ptm0_g, ptm0_b, 512)
    g1 = _gather_rows(p0a, idxk, 1024).reshape(K_, BN, c4)
    p1raw, p1sums = _kpconv(g1, nbp, xyz8, ptm1_kp, ptm1_W, 512)
    p1a = _bn_relu(p1raw, p1sums, ptm1_g, ptm1_b, 512)
    g2 = _gather_rows(p1a, idxk, 1024).reshape(K_, BN, c4)
    ptm = _kpconv_mask(g2, nbp, xyz8, ptm2_kp, ptm2_W, 512)

    r0, s0 = _kpconv(gfea, nbp, xyz8, body0_kp, body0_W, 512)
    o0 = _bn_relu_mask(r0, s0, body0_g, body0_b, ch_mask[0].T, ptm, 512)
    g3 = _gather_rows(o0, idxk, 256).reshape(K_, BN, COUT)
    r1, s1 = _kpconv(g3, nbp, xyz8, body1_kp, body1_W, 512)
    o1 = _bn_relu_mask(r1, s1, body1_g, body1_b, ch_mask[1].T, ptm, 512)

    trw, tsums = _tail_mm(o0, o1, tail_W[:COUT], tail_W[COUT:], tail_bias, 512)
    out = _tail_apply(trw, tsums, tail_g, tail_bb, feaT, 512)

    flops = _flops(ptm.reshape(B_, 1, N_), ch_mask).reshape(-1)
    total = jnp.float32(sum(B_ * N_ * K_ * COUT * ((CIN if i == 0 else COUT) + 1)
                            for i in range(NL)))
    return (out, flops, total)
